# baseline (speedup 1.0000x reference)
import jax
import jax.numpy as jnp
from jax import lax
from jax.experimental import pallas as pl
from jax.experimental.pallas import tpu as pltpu
from jax.experimental.pallas import tpu_sc as plsc

STATE_SIZE = 6
HIDDEN = 256
SEQ = 200
PAD_TOKEN = 1
EPS = 1e-12

NC = 2
NS = 16
NW = NC * NS
CHUNK = 80
PREP_BB = 256
REP = 16


def _combo_body(game_ref, pos_ref, gamma_ref, beta_ref, out_ref):
    x = game_ref[...][:, None, :] + pos_ref[...][None, :, :]
    mean = jnp.mean(x, axis=-1, keepdims=True)
    xc = x - mean
    var = jnp.mean(xc * xc, axis=-1, keepdims=True)
    y = xc * jax.lax.rsqrt(var + EPS)
    out_ref[...] = (y * gamma_ref[...][None, :, :] + beta_ref[...][None, :, :])[None]


def _idx_body(ids_ref, idx_ref):
    ids = ids_ref[...]
    s = lax.broadcasted_iota(jnp.int32, ids.shape, 1)
    row = lax.broadcasted_iota(jnp.int32, ids.shape, 0) + pl.program_id(0) * PREP_BB
    rep = (row // 128) % REP
    idx_ref[...] = ids * SEQ + s + rep * (STATE_SIZE * SEQ)


def _mask_body(ids_ref, mask_ref):
    mask_ref[...] = (ids_ref[...] != PAD_TOKEN).astype(jnp.int32)


def _sc_gather_body(combo_hbm, idx_hbm, out_hbm, idx_v,
                    buf0, buf1, buf2, buf3,
                    gsem0, gsem1, gsem2, gsem3,
                    wsem0, wsem1, wsem2, wsem3):
    n_chunks = idx_v.shape[0]
    rows_per_w = n_chunks * CHUNK
    wid = lax.axis_index("s") * NC + lax.axis_index("c")
    rows_base = wid * rows_per_w
    bufs = (buf0, buf1, buf2, buf3)
    gsems = (gsem0, gsem1, gsem2, gsem3)
    wsems = (wsem0, wsem1, wsem2, wsem3)

    pltpu.sync_copy(idx_hbm.at[wid], idx_v)
    pltpu.async_copy(combo_hbm.at[idx_v.at[0]], bufs[0], gsems[0])
    pltpu.async_copy(combo_hbm.at[idx_v.at[1]], bufs[1], gsems[1])

    @pl.loop(0, n_chunks, step=4)
    def _chunks(jj):
        for b in range(4):
            j = jj + b
            cur = b
            nxt = (b + 2) % 4
            pltpu.make_async_copy(combo_hbm.at[idx_v.at[j]], bufs[cur], gsems[cur]).wait()
            pltpu.async_copy(bufs[cur], out_hbm.at[pl.ds(rows_base + j * CHUNK, CHUNK)],
                             wsems[cur])

            @pl.when(j >= 2)
            def _wait_prev_write():
                pltpu.make_async_copy(
                    bufs[nxt], out_hbm.at[pl.ds(rows_base + (j - 2) * CHUNK, CHUNK)],
                    wsems[nxt]).wait()

            @pl.when(j + 2 < n_chunks)
            def _start_next():
                pltpu.async_copy(combo_hbm.at[idx_v.at[j + 2]], bufs[nxt], gsems[nxt])

    for b in (2, 3):
        pltpu.make_async_copy(
            bufs[b], out_hbm.at[pl.ds(rows_base + (n_chunks - 4 + b) * CHUNK, CHUNK)],
            wsems[b]).wait()


def kernel(input_ids, game_state_table, position_table, ln_gamma, ln_beta):
    batch, seq = input_ids.shape
    ids = input_ids.astype(jnp.int32)
    total_rows = batch * seq
    n_chunks = total_rows // (NW * CHUNK)

    combo = pl.pallas_call(
        _combo_body,
        grid=(REP,),
        out_shape=jax.ShapeDtypeStruct((REP, STATE_SIZE, SEQ, HIDDEN), jnp.float32),
        in_specs=[
            pl.BlockSpec((STATE_SIZE, HIDDEN), lambda r: (0, 0)),
            pl.BlockSpec((SEQ, HIDDEN), lambda r: (0, 0)),
            pl.BlockSpec((1, HIDDEN), lambda r: (0, 0)),
            pl.BlockSpec((1, HIDDEN), lambda r: (0, 0)),
        ],
        out_specs=pl.BlockSpec((1, STATE_SIZE, SEQ, HIDDEN), lambda r: (r, 0, 0, 0)),
    )(
        game_state_table,
        position_table[:SEQ],
        ln_gamma.reshape(1, HIDDEN),
        ln_beta.reshape(1, HIDDEN),
    )
    combo_flat = combo.reshape(REP * STATE_SIZE * SEQ, HIDDEN)

    flat_idx = pl.pallas_call(
        _idx_body,
        grid=(batch // PREP_BB,),
        out_shape=jax.ShapeDtypeStruct((batch, seq), jnp.int32),
        in_specs=[pl.BlockSpec((PREP_BB, seq), lambda i: (i, 0))],
        out_specs=pl.BlockSpec((PREP_BB, seq), lambda i: (i, 0)),
    )(ids)

    idx3 = flat_idx.reshape(NW, n_chunks, CHUNK)

    sc_gather = pl.kernel(
        _sc_gather_body,
        out_type=jax.ShapeDtypeStruct((total_rows, HIDDEN), jnp.float32),
        mesh=plsc.VectorSubcoreMesh(core_axis_name="c", subcore_axis_name="s"),
        scratch_types=(
            [pltpu.VMEM((n_chunks, CHUNK), jnp.int32)]
            + [pltpu.VMEM((CHUNK, HIDDEN), jnp.float32)] * 4
            + [pltpu.SemaphoreType.DMA] * 8
        ),
    )
    out_flat = sc_gather(combo_flat, idx3)

    mask = pl.pallas_call(
        _mask_body,
        grid=(batch // PREP_BB,),
        out_shape=jax.ShapeDtypeStruct((batch, seq), jnp.int32),
        in_specs=[pl.BlockSpec((PREP_BB, seq), lambda i: (i, 0))],
        out_specs=pl.BlockSpec((PREP_BB, seq), lambda i: (i, 0)),
    )(ids)

    return out_flat.reshape(batch, seq, HIDDEN), mask


# speedup vs baseline: 1.0023x; 1.0023x over previous
import jax
import jax.numpy as jnp
from jax import lax
from jax.experimental import pallas as pl
from jax.experimental.pallas import tpu as pltpu
from jax.experimental.pallas import tpu_sc as plsc

STATE_SIZE = 6
HIDDEN = 256
SEQ = 200
PAD_TOKEN = 1
EPS = 1e-12

NC = 2
NS = 16
NW = NC * NS
CHUNK = 80
PREP_BB = 256
REP = 8


def _combo_body(game_ref, pos_ref, gamma_ref, beta_ref, out_ref):
    x = game_ref[...][:, None, :] + pos_ref[...][None, :, :]
    mean = jnp.mean(x, axis=-1, keepdims=True)
    xc = x - mean
    var = jnp.mean(xc * xc, axis=-1, keepdims=True)
    y = xc * jax.lax.rsqrt(var + EPS)
    out_ref[...] = (y * gamma_ref[...][None, :, :] + beta_ref[...][None, :, :])[None]


def _idx_body(ids_ref, idx_ref):
    ids = ids_ref[...]
    s = lax.broadcasted_iota(jnp.int32, ids.shape, 1)
    row = lax.broadcasted_iota(jnp.int32, ids.shape, 0) + pl.program_id(0) * PREP_BB
    rep = (row // 128) % REP
    idx_ref[...] = ids * SEQ + s + rep * (STATE_SIZE * SEQ)


def _mask_body(ids_ref, mask_ref):
    mask_ref[...] = (ids_ref[...] != PAD_TOKEN).astype(jnp.int32)


def _sc_gather_body(combo_hbm, idx_hbm, out_hbm, idx_v,
                    buf0, buf1, buf2, buf3,
                    gsem0, gsem1, gsem2, gsem3,
                    wsem0, wsem1, wsem2, wsem3):
    n_chunks = idx_v.shape[0]
    rows_per_w = n_chunks * CHUNK
    wid = lax.axis_index("s") * NC + lax.axis_index("c")
    rows_base = wid * rows_per_w
    bufs = (buf0, buf1, buf2, buf3)
    gsems = (gsem0, gsem1, gsem2, gsem3)
    wsems = (wsem0, wsem1, wsem2, wsem3)

    pltpu.sync_copy(idx_hbm.at[wid], idx_v)
    pltpu.async_copy(combo_hbm.at[idx_v.at[0]], bufs[0], gsems[0])
    pltpu.async_copy(combo_hbm.at[idx_v.at[1]], bufs[1], gsems[1])

    @pl.loop(0, n_chunks, step=4)
    def _chunks(jj):
        for b in range(4):
            j = jj + b
            cur = b
            nxt = (b + 2) % 4
            pltpu.make_async_copy(combo_hbm.at[idx_v.at[j]], bufs[cur], gsems[cur]).wait()
            pltpu.async_copy(bufs[cur], out_hbm.at[pl.ds(rows_base + j * CHUNK, CHUNK)],
                             wsems[cur])

            @pl.when(j >= 2)
            def _wait_prev_write():
                pltpu.make_async_copy(
                    bufs[nxt], out_hbm.at[pl.ds(rows_base + (j - 2) * CHUNK, CHUNK)],
                    wsems[nxt]).wait()

            @pl.when(j + 2 < n_chunks)
            def _start_next():
                pltpu.async_copy(combo_hbm.at[idx_v.at[j + 2]], bufs[nxt], gsems[nxt])

    for b in (2, 3):
        pltpu.make_async_copy(
            bufs[b], out_hbm.at[pl.ds(rows_base + (n_chunks - 4 + b) * CHUNK, CHUNK)],
            wsems[b]).wait()


def kernel(input_ids, game_state_table, position_table, ln_gamma, ln_beta):
    batch, seq = input_ids.shape
    ids = input_ids.astype(jnp.int32)
    total_rows = batch * seq
    n_chunks = total_rows // (NW * CHUNK)

    combo = pl.pallas_call(
        _combo_body,
        grid=(REP,),
        out_shape=jax.ShapeDtypeStruct((REP, STATE_SIZE, SEQ, HIDDEN), jnp.float32),
        in_specs=[
            pl.BlockSpec((STATE_SIZE, HIDDEN), lambda r: (0, 0)),
            pl.BlockSpec((SEQ, HIDDEN), lambda r: (0, 0)),
            pl.BlockSpec((1, HIDDEN), lambda r: (0, 0)),
            pl.BlockSpec((1, HIDDEN), lambda r: (0, 0)),
        ],
        out_specs=pl.BlockSpec((1, STATE_SIZE, SEQ, HIDDEN), lambda r: (r, 0, 0, 0)),
    )(
        game_state_table,
        position_table[:SEQ],
        ln_gamma.reshape(1, HIDDEN),
        ln_beta.reshape(1, HIDDEN),
    )
    combo_flat = combo.reshape(REP * STATE_SIZE * SEQ, HIDDEN)

    flat_idx = pl.pallas_call(
        _idx_body,
        grid=(batch // PREP_BB,),
        out_shape=jax.ShapeDtypeStruct((batch, seq), jnp.int32),
        in_specs=[pl.BlockSpec((PREP_BB, seq), lambda i: (i, 0))],
        out_specs=pl.BlockSpec((PREP_BB, seq), lambda i: (i, 0)),
    )(ids)

    idx3 = flat_idx.reshape(NW, n_chunks, CHUNK)

    sc_gather = pl.kernel(
        _sc_gather_body,
        out_type=jax.ShapeDtypeStruct((total_rows, HIDDEN), jnp.float32),
        mesh=plsc.VectorSubcoreMesh(core_axis_name="c", subcore_axis_name="s"),
        scratch_types=(
            [pltpu.VMEM((n_chunks, CHUNK), jnp.int32)]
            + [pltpu.VMEM((CHUNK, HIDDEN), jnp.float32)] * 4
            + [pltpu.SemaphoreType.DMA] * 8
        ),
    )
    out_flat = sc_gather(combo_flat, idx3)

    mask = pl.pallas_call(
        _mask_body,
        grid=(batch // PREP_BB,),
        out_shape=jax.ShapeDtypeStruct((batch, seq), jnp.int32),
        in_specs=[pl.BlockSpec((PREP_BB, seq), lambda i: (i, 0))],
        out_specs=pl.BlockSpec((PREP_BB, seq), lambda i: (i, 0)),
    )(ids)

    return out_flat.reshape(batch, seq, HIDDEN), mask


# speedup vs baseline: 1.0090x; 1.0067x over previous
import jax
import jax.numpy as jnp
from jax import lax
from jax.experimental import pallas as pl
from jax.experimental.pallas import tpu as pltpu
from jax.experimental.pallas import tpu_sc as plsc

STATE_SIZE = 6
HIDDEN = 256
SEQ = 200
PAD_TOKEN = 1
EPS = 1e-12

NC = 2
NS = 16
NW = NC * NS
CHUNK = 80
PREP_BB = 256
REP = 8


def _combo_body(game_ref, pos_ref, gamma_ref, beta_ref, out_ref):
    x = game_ref[...][:, None, :] + pos_ref[...][None, :, :]
    mean = jnp.mean(x, axis=-1, keepdims=True)
    xc = x - mean
    var = jnp.mean(xc * xc, axis=-1, keepdims=True)
    y = xc * jax.lax.rsqrt(var + EPS)
    out_ref[...] = (y * gamma_ref[...][None, :, :] + beta_ref[...][None, :, :])[None]


def _idx_body(ids_ref, idx_ref):
    ids = ids_ref[...]
    s = lax.broadcasted_iota(jnp.int32, ids.shape, 1)
    row = lax.broadcasted_iota(jnp.int32, ids.shape, 0) + pl.program_id(0) * PREP_BB
    rep = (row // 128) % REP
    idx_ref[...] = ids * SEQ + s + rep * (STATE_SIZE * SEQ)


def _mask_body(ids_ref, mask_ref):
    mask_ref[...] = (ids_ref[...] != PAD_TOKEN).astype(jnp.int32)


def _sc_gather_body(combo_hbm, idx_hbm, out_hbm, idx_v,
                    buf0, buf1, buf2, buf3,
                    gsem0, gsem1, gsem2, gsem3,
                    wsem0, wsem1, wsem2, wsem3):
    n_chunks = idx_v.shape[0]
    rows_per_w = n_chunks * CHUNK
    wid = lax.axis_index("s") * NC + lax.axis_index("c")
    rows_base = wid * rows_per_w
    bufs = (buf0, buf1, buf2, buf3)
    gsems = (gsem0, gsem1, gsem2, gsem3)
    wsems = (wsem0, wsem1, wsem2, wsem3)

    pltpu.sync_copy(idx_hbm.at[wid], idx_v)

    phase = (wid // 8) * (n_chunks // 4)

    def _eff(j):
        return lax.rem(j + phase, n_chunks)

    pltpu.async_copy(combo_hbm.at[idx_v.at[_eff(0)]], bufs[0], gsems[0])
    pltpu.async_copy(combo_hbm.at[idx_v.at[_eff(1)]], bufs[1], gsems[1])

    @pl.loop(0, n_chunks, step=4)
    def _chunks(jj):
        for b in range(4):
            j = jj + b
            cur = b
            nxt = (b + 2) % 4
            je = _eff(j)
            pltpu.make_async_copy(combo_hbm.at[idx_v.at[je]], bufs[cur], gsems[cur]).wait()
            pltpu.async_copy(bufs[cur], out_hbm.at[pl.ds(rows_base + je * CHUNK, CHUNK)],
                             wsems[cur])

            @pl.when(j >= 2)
            def _wait_prev_write():
                pltpu.make_async_copy(
                    bufs[nxt], out_hbm.at[pl.ds(rows_base + _eff(j - 2) * CHUNK, CHUNK)],
                    wsems[nxt]).wait()

            @pl.when(j + 2 < n_chunks)
            def _start_next():
                pltpu.async_copy(combo_hbm.at[idx_v.at[_eff(j + 2)]], bufs[nxt], gsems[nxt])

    for b in (2, 3):
        pltpu.make_async_copy(
            bufs[b],
            out_hbm.at[pl.ds(rows_base + _eff(n_chunks - 4 + b) * CHUNK, CHUNK)],
            wsems[b]).wait()


def kernel(input_ids, game_state_table, position_table, ln_gamma, ln_beta):
    batch, seq = input_ids.shape
    ids = input_ids.astype(jnp.int32)
    total_rows = batch * seq
    n_chunks = total_rows // (NW * CHUNK)

    combo = pl.pallas_call(
        _combo_body,
        grid=(REP,),
        out_shape=jax.ShapeDtypeStruct((REP, STATE_SIZE, SEQ, HIDDEN), jnp.float32),
        in_specs=[
            pl.BlockSpec((STATE_SIZE, HIDDEN), lambda r: (0, 0)),
            pl.BlockSpec((SEQ, HIDDEN), lambda r: (0, 0)),
            pl.BlockSpec((1, HIDDEN), lambda r: (0, 0)),
            pl.BlockSpec((1, HIDDEN), lambda r: (0, 0)),
        ],
        out_specs=pl.BlockSpec((1, STATE_SIZE, SEQ, HIDDEN), lambda r: (r, 0, 0, 0)),
    )(
        game_state_table,
        position_table[:SEQ],
        ln_gamma.reshape(1, HIDDEN),
        ln_beta.reshape(1, HIDDEN),
    )
    combo_flat = combo.reshape(REP * STATE_SIZE * SEQ, HIDDEN)

    flat_idx = pl.pallas_call(
        _idx_body,
        grid=(batch // PREP_BB,),
        out_shape=jax.ShapeDtypeStruct((batch, seq), jnp.int32),
        in_specs=[pl.BlockSpec((PREP_BB, seq), lambda i: (i, 0))],
        out_specs=pl.BlockSpec((PREP_BB, seq), lambda i: (i, 0)),
    )(ids)

    idx3 = flat_idx.reshape(NW, n_chunks, CHUNK)

    sc_gather = pl.kernel(
        _sc_gather_body,
        out_type=jax.ShapeDtypeStruct((total_rows, HIDDEN), jnp.float32),
        mesh=plsc.VectorSubcoreMesh(core_axis_name="c", subcore_axis_name="s"),
        scratch_types=(
            [pltpu.VMEM((n_chunks, CHUNK), jnp.int32)]
            + [pltpu.VMEM((CHUNK, HIDDEN), jnp.float32)] * 4
            + [pltpu.SemaphoreType.DMA] * 8
        ),
    )
    out_flat = sc_gather(combo_flat, idx3)

    mask = pl.pallas_call(
        _mask_body,
        grid=(batch // PREP_BB,),
        out_shape=jax.ShapeDtypeStruct((batch, seq), jnp.int32),
        in_specs=[pl.BlockSpec((PREP_BB, seq), lambda i: (i, 0))],
        out_specs=pl.BlockSpec((PREP_BB, seq), lambda i: (i, 0)),
    )(ids)

    return out_flat.reshape(batch, seq, HIDDEN), mask
